# baseline (speedup 1.0000x reference)
import jax
import jax.numpy as jnp
from jax import lax
from jax.experimental import pallas as pl
from jax.experimental.pallas import tpu as pltpu
from jax.experimental.pallas import tpu_sc as plsc

N = 20000
IOU_T = 0.5
L = 16
NSUB = 16
NP = 20480
B = NSUB * L
NB = NP // B
M = NP // B
CW = M * L


def _make_nms(interpret=False):
    mesh = plsc.VectorSubcoreMesh(
        core_axis_name="c", subcore_axis_name="s", num_cores=1, num_subcores=NSUB
    )
    O1, O2, O3, O4 = NP, 2 * NP, 3 * NP, 4 * NP
    S1, S2, S3, S4 = CW, 2 * CW, 3 * CW, 4 * CW

    def body(x1_hbm, y1_hbm, x2_hbm, y2_hbm, sc_hbm,
             x1s_hbm, y1s_hbm, x2s_hbm, y2s_hbm, pos_hbm, gmn_hbm, gmx_hbm,
             out_hbm,
             fx, sox, pos_v, stage, gmn, gmx, supwin, flagwin, blkio,
             delta, idxb, sco, sem, sem2, sup_sh):
        w = lax.axis_index("s")
        ob = w * CW
        pltpu.sync_copy(x1_hbm, fx.at[pl.ds(0, NP)])
        pltpu.sync_copy(y1_hbm, fx.at[pl.ds(O1, NP)])
        pltpu.sync_copy(x2_hbm, fx.at[pl.ds(O2, NP)])
        pltpu.sync_copy(y2_hbm, fx.at[pl.ds(O3, NP)])
        pltpu.sync_copy(x1s_hbm.at[pl.ds(ob, CW)], sox.at[pl.ds(0, CW)])
        pltpu.sync_copy(y1s_hbm.at[pl.ds(ob, CW)], sox.at[pl.ds(S1, CW)])
        pltpu.sync_copy(x2s_hbm.at[pl.ds(ob, CW)], sox.at[pl.ds(S2, CW)])
        pltpu.sync_copy(y2s_hbm.at[pl.ds(ob, CW)], sox.at[pl.ds(S3, CW)])
        pltpu.sync_copy(pos_hbm.at[pl.ds(ob, CW)], pos_v)
        pltpu.sync_copy(gmn_hbm.at[pl.ds(w * M, M)], gmn)
        pltpu.sync_copy(gmx_hbm.at[pl.ds(w * M, M)], gmx)

        iota = lax.iota(jnp.int32, L)

        def init_fx(v, mw):
            o = v * L
            x1 = fx[pl.ds(o, L)]
            x2 = fx[pl.ds(O2 + o, L)]
            fx[pl.ds(O4 + o, L)] = (x2 - x1) * (fx[pl.ds(O3 + o, L)] - fx[pl.ds(O1 + o, L)])
            return jnp.maximum(mw, x2 - x1)
        mwv = lax.fori_loop(0, NP // L, init_fx, jnp.zeros((L,), jnp.float32))
        maxw = mwv[0]
        for _j in range(1, L):
            maxw = jnp.maximum(maxw, mwv[_j])

        def init_sox(v, _):
            o = v * L
            x1 = sox[pl.ds(o, L)]
            x2 = sox[pl.ds(S2 + o, L)]
            sox[pl.ds(S4 + o, L)] = (x2 - x1) * (sox[pl.ds(S3 + o, L)] - sox[pl.ds(S1 + o, L)])
            stage[pl.ds(o, L)] = jnp.zeros((L,), jnp.float32)
            return 0
        lax.fori_loop(0, M, init_sox, 0)

        def init_io(v, _):
            blkio[pl.ds(v * L, L)] = iota + v * L
            return 0
        lax.fori_loop(0, B // L, init_io, 0)

        pltpu.sync_copy(stage, sup_sh.at[pl.ds(ob, CW)])
        plsc.subcore_barrier()

        def kblock(k, _):
            kb = k * B
            pltpu.sync_copy(sup_sh.at[pl.ds(kb, B)], supwin)

            def init_fw(v, _):
                sl = pl.ds(v * L, L)
                flagwin[sl] = jnp.where(supwin[sl] > 0.0,
                                        jnp.zeros((L,), jnp.float32),
                                        jnp.full((L,), 1.0, jnp.float32))
                return 0
            lax.fori_loop(0, B // L, init_fw, 0)

            def gbody(gp, _):
                o = kb + gp * L
                x1g = fx[pl.ds(o, L)]
                y1g = fx[pl.ds(O1 + o, L)]
                x2g = fx[pl.ds(O2 + o, L)]
                y2g = fx[pl.ds(O3 + o, L)]
                arg = fx[pl.ds(O4 + o, L)]
                for l in range(L):
                    alv = flagwin[pl.ds(gp * L, L)]
                    @pl.when(alv[l] > 0.5)
                    def _(l=l):
                        bx1 = x1g[l]
                        by1 = y1g[l]
                        bx2 = x2g[l]
                        by2 = y2g[l]
                        ba = arg[l]
                        p = gp * L + l
                        rpos = kb + p

                        @plsc.parallel_loop(gp, B // L, unroll=2)
                        def vblk(v):
                            obk = kb + v * L
                            xx1 = jnp.maximum(bx1, fx[pl.ds(obk, L)])
                            yy1 = jnp.maximum(by1, fx[pl.ds(O1 + obk, L)])
                            xx2 = jnp.minimum(bx2, fx[pl.ds(O2 + obk, L)])
                            yy2 = jnp.minimum(by2, fx[pl.ds(O3 + obk, L)])
                            iw = jnp.maximum(xx2 - xx1, 0.0)
                            ih = jnp.maximum(yy2 - yy1, 0.0)
                            inter = iw * ih
                            union = jnp.maximum(ba + fx[pl.ds(O4 + obk, L)] - inter, 1e-6)
                            over = (inter > IOU_T * union) & (blkio[pl.ds(v * L, L)] > p)
                            fw = flagwin[pl.ds(v * L, L)]
                            flagwin[pl.ds(v * L, L)] = jnp.where(over, 0.0, fw)

                        xr_lo = bx1 - maxw - 1.0
                        lo_acc = jnp.zeros((L,), jnp.int32)
                        hi_acc = jnp.zeros((L,), jnp.int32)
                        one = jnp.full((L,), 1, jnp.int32)
                        zero = jnp.zeros((L,), jnp.int32)
                        for t in range(M // L):
                            lo_acc = lo_acc + jnp.where(gmx[pl.ds(t * L, L)] < xr_lo, one, zero)
                            hi_acc = hi_acc + jnp.where(gmn[pl.ds(t * L, L)] <= bx2, one, zero)
                        mlo = lo_acc[0]
                        mhi = hi_acc[0]
                        for _j in range(1, L):
                            mlo = mlo + lo_acc[_j]
                            mhi = mhi + hi_acc[_j]

                        @plsc.parallel_loop(mlo, mhi, unroll=4)
                        def mbody(v2):
                            oc = v2 * L
                            xx1 = jnp.maximum(bx1, sox[pl.ds(oc, L)])
                            yy1 = jnp.maximum(by1, sox[pl.ds(S1 + oc, L)])
                            xx2 = jnp.minimum(bx2, sox[pl.ds(S2 + oc, L)])
                            yy2 = jnp.minimum(by2, sox[pl.ds(S3 + oc, L)])
                            iw = jnp.maximum(xx2 - xx1, 0.0)
                            ih = jnp.maximum(yy2 - yy1, 0.0)
                            inter = iw * ih
                            union = jnp.maximum(ba + sox[pl.ds(S4 + oc, L)] - inter, 1e-6)
                            over = (inter > IOU_T * union) & (pos_v[pl.ds(oc, L)] != rpos)
                            st = stage[pl.ds(oc, L)]
                            stage[pl.ds(oc, L)] = jnp.where(over, jnp.full((L,), 1.0, jnp.float32), st)
                return 0
            lax.fori_loop(0, B // L, gbody, 0)

            sl = pl.ds(w * L, L)
            delta[pl.ds(0, L)] = jnp.where(supwin[sl] > 0.0,
                                           jnp.zeros((L,), jnp.float32),
                                           jnp.full((L,), 1.0, jnp.float32)) - flagwin[sl]
            idxb[pl.ds(0, L)] = iota + (kb + w * L)
            cp1 = pltpu.async_copy(delta, sup_sh.at[idxb], sem, add=True)
            cp2 = pltpu.async_copy(stage, sup_sh.at[pos_v], sem2, add=True)
            cp1.wait()
            cp2.wait()
            plsc.subcore_barrier()
            return 0

        lax.fori_loop(0, NB, kblock, 0)

        pltpu.sync_copy(sc_hbm.at[pl.ds(ob, CW)], sco)
        pltpu.sync_copy(sup_sh.at[pl.ds(ob, CW)], stage)

        def outv(v2, _):
            o = v2 * L
            keep = jnp.where(stage[pl.ds(o, L)] > 0.0,
                             jnp.zeros((L,), jnp.float32),
                             jnp.full((L,), 1.0, jnp.float32))
            sco[pl.ds(o, L)] = sco[pl.ds(o, L)] * keep
            return 0
        lax.fori_loop(0, M, outv, 0)
        pltpu.sync_copy(sco, out_hbm.at[pl.ds(ob, CW)])

    return pl.kernel(
        body,
        out_type=jax.ShapeDtypeStruct((NP,), jnp.float32),
        mesh=mesh,
        scratch_types=[
            pltpu.VMEM((5 * NP,), jnp.float32),
            pltpu.VMEM((5 * CW,), jnp.float32),
            pltpu.VMEM((CW,), jnp.int32),
            pltpu.VMEM((CW,), jnp.float32),
            pltpu.VMEM((M,), jnp.float32),
            pltpu.VMEM((M,), jnp.float32),
            pltpu.VMEM((B,), jnp.float32),
            pltpu.VMEM((B,), jnp.float32),
            pltpu.VMEM((B,), jnp.int32),
            pltpu.VMEM((L,), jnp.float32),
            pltpu.VMEM((L,), jnp.int32),
            pltpu.VMEM((CW,), jnp.float32),
            pltpu.SemaphoreType.DMA,
            pltpu.SemaphoreType.DMA,
            pltpu.VMEM_SHARED((NP,), jnp.float32),
        ],
        interpret=interpret,
    )


_nms = _make_nms()


@jax.jit
def kernel(boxes, scores):
    order = jnp.argsort(-scores)
    boxes_s = jnp.take(boxes, order, axis=0)
    scores_s = jnp.take(scores, order, axis=0)
    pad = NP - N
    x1 = jnp.pad(boxes_s[:, 0], (0, pad))
    y1 = jnp.pad(boxes_s[:, 1], (0, pad))
    x2 = jnp.pad(boxes_s[:, 2], (0, pad))
    y2 = jnp.pad(boxes_s[:, 3], (0, pad))
    sc = jnp.pad(scores_s, (0, pad))

    xr = jnp.argsort(x1)

    def rearr(a):
        return a.reshape(M, NSUB, L).transpose(1, 0, 2).reshape(NP)

    x1r = x1[xr]
    x1s = rearr(x1r)
    y1s = rearr(y1[xr])
    x2s = rearr(x2[xr])
    y2s = rearr(y2[xr])
    pos_s = rearr(xr.astype(jnp.int32))
    gm = x1r.reshape(NP // L, L)
    gmn = gm[:, 0].reshape(M, NSUB).T.reshape(NP // L)
    gmx = gm[:, L - 1].reshape(M, NSUB).T.reshape(NP // L)

    kept = _nms(x1, y1, x2, y2, sc,
                x1s, y1s, x2s, y2s, pos_s, gmn, gmx)
    return kept[:N]


# speedup vs baseline: 1.0351x; 1.0351x over previous
import jax
import jax.numpy as jnp
from jax import lax
from jax.experimental import pallas as pl
from jax.experimental.pallas import tpu as pltpu
from jax.experimental.pallas import tpu_sc as plsc

N = 20000
IOU_T = 0.5
L = 16
NSUB = 16
NP = 20480
B = NSUB * L
NB = NP // B
M = NP // B
CW = M * L


def _make_nms(interpret=False):
    mesh = plsc.VectorSubcoreMesh(
        core_axis_name="c", subcore_axis_name="s", num_cores=1, num_subcores=NSUB
    )
    O1, O2, O3, O4 = NP, 2 * NP, 3 * NP, 4 * NP
    S1, S2, S3, S4 = CW, 2 * CW, 3 * CW, 4 * CW

    def body(x1_hbm, y1_hbm, x2_hbm, y2_hbm, sc_hbm,
             x1s_hbm, y1s_hbm, x2s_hbm, y2s_hbm, pos_hbm, rlo_hbm, rhi_hbm,
             out_hbm,
             fx, sox, pos_v, stage, rlo_v, rhi_v, supwin, flagwin, blkio,
             delta, idxb, sco, sem, sem2, sup_sh):
        w = lax.axis_index("s")
        ob = w * CW
        pltpu.sync_copy(x1_hbm, fx.at[pl.ds(0, NP)])
        pltpu.sync_copy(y1_hbm, fx.at[pl.ds(O1, NP)])
        pltpu.sync_copy(x2_hbm, fx.at[pl.ds(O2, NP)])
        pltpu.sync_copy(y2_hbm, fx.at[pl.ds(O3, NP)])
        pltpu.sync_copy(x1s_hbm.at[pl.ds(ob, CW)], sox.at[pl.ds(0, CW)])
        pltpu.sync_copy(y1s_hbm.at[pl.ds(ob, CW)], sox.at[pl.ds(S1, CW)])
        pltpu.sync_copy(x2s_hbm.at[pl.ds(ob, CW)], sox.at[pl.ds(S2, CW)])
        pltpu.sync_copy(y2s_hbm.at[pl.ds(ob, CW)], sox.at[pl.ds(S3, CW)])
        pltpu.sync_copy(pos_hbm.at[pl.ds(ob, CW)], pos_v)

        iota = lax.iota(jnp.int32, L)

        def init_fx(v, mw):
            o = v * L
            x1 = fx[pl.ds(o, L)]
            x2 = fx[pl.ds(O2 + o, L)]
            fx[pl.ds(O4 + o, L)] = (x2 - x1) * (fx[pl.ds(O3 + o, L)] - fx[pl.ds(O1 + o, L)])
            return mw
        lax.fori_loop(0, NP // L, init_fx, jnp.zeros((L,), jnp.float32))

        def init_sox(v, _):
            o = v * L
            x1 = sox[pl.ds(o, L)]
            x2 = sox[pl.ds(S2 + o, L)]
            sox[pl.ds(S4 + o, L)] = (x2 - x1) * (sox[pl.ds(S3 + o, L)] - sox[pl.ds(S1 + o, L)])
            stage[pl.ds(o, L)] = jnp.zeros((L,), jnp.float32)
            return 0
        lax.fori_loop(0, M, init_sox, 0)

        def init_io(v, _):
            blkio[pl.ds(v * L, L)] = iota + v * L
            return 0
        lax.fori_loop(0, B // L, init_io, 0)

        pltpu.sync_copy(stage, sup_sh.at[pl.ds(ob, CW)])
        plsc.subcore_barrier()

        def kblock(k, _):
            kb = k * B
            pltpu.sync_copy(sup_sh.at[pl.ds(kb, B)], supwin)
            pltpu.sync_copy(rlo_hbm.at[pl.ds(kb, B)], rlo_v)
            pltpu.sync_copy(rhi_hbm.at[pl.ds(kb, B)], rhi_v)

            def init_fw(v, _):
                sl = pl.ds(v * L, L)
                flagwin[sl] = jnp.where(supwin[sl] > 0.0,
                                        jnp.zeros((L,), jnp.float32),
                                        jnp.full((L,), 1.0, jnp.float32))
                return 0
            lax.fori_loop(0, B // L, init_fw, 0)

            def gbody(gp, _):
                o = kb + gp * L
                x1g = fx[pl.ds(o, L)]
                y1g = fx[pl.ds(O1 + o, L)]
                x2g = fx[pl.ds(O2 + o, L)]
                y2g = fx[pl.ds(O3 + o, L)]
                arg = fx[pl.ds(O4 + o, L)]
                rlog = rlo_v[pl.ds(gp * L, L)]
                rhig = rhi_v[pl.ds(gp * L, L)]
                for l in range(L):
                    alv = flagwin[pl.ds(gp * L, L)]
                    @pl.when(alv[l] > 0.5)
                    def _(l=l):
                        bx1 = x1g[l]
                        by1 = y1g[l]
                        bx2 = x2g[l]
                        by2 = y2g[l]
                        ba = arg[l]
                        p = gp * L + l
                        rpos = kb + p

                        @plsc.parallel_loop(gp, B // L, unroll=2)
                        def vblk(v):
                            obk = kb + v * L
                            xx1 = jnp.maximum(bx1, fx[pl.ds(obk, L)])
                            yy1 = jnp.maximum(by1, fx[pl.ds(O1 + obk, L)])
                            xx2 = jnp.minimum(bx2, fx[pl.ds(O2 + obk, L)])
                            yy2 = jnp.minimum(by2, fx[pl.ds(O3 + obk, L)])
                            iw = jnp.maximum(xx2 - xx1, 0.0)
                            ih = jnp.maximum(yy2 - yy1, 0.0)
                            inter = iw * ih
                            union = jnp.maximum(ba + fx[pl.ds(O4 + obk, L)] - inter, 1e-6)
                            over = (inter > IOU_T * union) & (blkio[pl.ds(v * L, L)] > p)
                            fw = flagwin[pl.ds(v * L, L)]
                            flagwin[pl.ds(v * L, L)] = jnp.where(over, 0.0, fw)

                        glo = lax.shift_right_logical(rlog[l], 4)
                        ghi = lax.shift_right_logical(rhig[l] + 15, 4)
                        mlo = lax.shift_right_logical(glo + 15 - w, 4)
                        mhi = lax.shift_right_logical(ghi + 15 - w, 4)

                        @plsc.parallel_loop(mlo, mhi, unroll=4)
                        def mbody(v2):
                            oc = v2 * L
                            xx1 = jnp.maximum(bx1, sox[pl.ds(oc, L)])
                            yy1 = jnp.maximum(by1, sox[pl.ds(S1 + oc, L)])
                            xx2 = jnp.minimum(bx2, sox[pl.ds(S2 + oc, L)])
                            yy2 = jnp.minimum(by2, sox[pl.ds(S3 + oc, L)])
                            iw = jnp.maximum(xx2 - xx1, 0.0)
                            ih = jnp.maximum(yy2 - yy1, 0.0)
                            inter = iw * ih
                            union = jnp.maximum(ba + sox[pl.ds(S4 + oc, L)] - inter, 1e-6)
                            over = (inter > IOU_T * union) & (pos_v[pl.ds(oc, L)] != rpos)
                            st = stage[pl.ds(oc, L)]
                            stage[pl.ds(oc, L)] = jnp.where(over, jnp.full((L,), 1.0, jnp.float32), st)
                return 0
            lax.fori_loop(0, B // L, gbody, 0)

            sl = pl.ds(w * L, L)
            delta[pl.ds(0, L)] = jnp.where(supwin[sl] > 0.0,
                                           jnp.zeros((L,), jnp.float32),
                                           jnp.full((L,), 1.0, jnp.float32)) - flagwin[sl]
            idxb[pl.ds(0, L)] = iota + (kb + w * L)
            cp1 = pltpu.async_copy(delta, sup_sh.at[idxb], sem, add=True)
            cp2 = pltpu.async_copy(stage, sup_sh.at[pos_v], sem2, add=True)
            cp1.wait()
            cp2.wait()
            plsc.subcore_barrier()
            return 0

        lax.fori_loop(0, NB, kblock, 0)

        pltpu.sync_copy(sc_hbm.at[pl.ds(ob, CW)], sco)
        pltpu.sync_copy(sup_sh.at[pl.ds(ob, CW)], stage)

        def outv(v2, _):
            o = v2 * L
            keep = jnp.where(stage[pl.ds(o, L)] > 0.0,
                             jnp.zeros((L,), jnp.float32),
                             jnp.full((L,), 1.0, jnp.float32))
            sco[pl.ds(o, L)] = sco[pl.ds(o, L)] * keep
            return 0
        lax.fori_loop(0, M, outv, 0)
        pltpu.sync_copy(sco, out_hbm.at[pl.ds(ob, CW)])

    return pl.kernel(
        body,
        out_type=jax.ShapeDtypeStruct((NP,), jnp.float32),
        mesh=mesh,
        scratch_types=[
            pltpu.VMEM((5 * NP,), jnp.float32),
            pltpu.VMEM((5 * CW,), jnp.float32),
            pltpu.VMEM((CW,), jnp.int32),
            pltpu.VMEM((CW,), jnp.float32),
            pltpu.VMEM((B,), jnp.int32),
            pltpu.VMEM((B,), jnp.int32),
            pltpu.VMEM((B,), jnp.float32),
            pltpu.VMEM((B,), jnp.float32),
            pltpu.VMEM((B,), jnp.int32),
            pltpu.VMEM((L,), jnp.float32),
            pltpu.VMEM((L,), jnp.int32),
            pltpu.VMEM((CW,), jnp.float32),
            pltpu.SemaphoreType.DMA,
            pltpu.SemaphoreType.DMA,
            pltpu.VMEM_SHARED((NP,), jnp.float32),
        ],
        interpret=interpret,
    )


_nms = _make_nms()


@jax.jit
def kernel(boxes, scores):
    order = jnp.argsort(-scores)
    boxes_s = jnp.take(boxes, order, axis=0)
    scores_s = jnp.take(scores, order, axis=0)
    pad = NP - N
    x1 = jnp.pad(boxes_s[:, 0], (0, pad))
    y1 = jnp.pad(boxes_s[:, 1], (0, pad))
    x2 = jnp.pad(boxes_s[:, 2], (0, pad))
    y2 = jnp.pad(boxes_s[:, 3], (0, pad))
    sc = jnp.pad(scores_s, (0, pad))

    xr = jnp.argsort(x1)

    def rearr(a):
        return a.reshape(M, NSUB, L).transpose(1, 0, 2).reshape(NP)

    x1r = x1[xr]
    x1s = rearr(x1r)
    y1s = rearr(y1[xr])
    x2s = rearr(x2[xr])
    y2s = rearr(y2[xr])
    pos_s = rearr(xr.astype(jnp.int32))
    maxw = jnp.max(x2 - x1)
    rlo = jnp.searchsorted(x1r, x1 - maxw - 1.0, side="left").astype(jnp.int32)
    rhi = jnp.searchsorted(x1r, x2, side="right").astype(jnp.int32)

    kept = _nms(x1, y1, x2, y2, sc,
                x1s, y1s, x2s, y2s, pos_s, rlo, rhi)
    return kept[:N]


# speedup vs baseline: 1.0751x; 1.0386x over previous
import jax
import jax.numpy as jnp
from jax import lax
from jax.experimental import pallas as pl
from jax.experimental.pallas import tpu as pltpu
from jax.experimental.pallas import tpu_sc as plsc

N = 20000
IOU_T = 0.5
L = 16
NSUB = 16
NP = 20480
B = NSUB * L
NB = NP // B
M = NP // B


def _make_nms(interpret=False):
    mesh = plsc.VectorSubcoreMesh(
        core_axis_name="c", subcore_axis_name="s", num_cores=1, num_subcores=NSUB
    )
    O1, O2, O3, O4 = NP, 2 * NP, 3 * NP, 4 * NP

    def body(x1_hbm, y1_hbm, x2_hbm, y2_hbm, sc_hbm, out_hbm,
             fx, al, flagwin, blkio, sco, flag_sh):
        w = lax.axis_index("s")
        pltpu.sync_copy(x1_hbm, fx.at[pl.ds(0, NP)])
        pltpu.sync_copy(y1_hbm, fx.at[pl.ds(O1, NP)])
        pltpu.sync_copy(x2_hbm, fx.at[pl.ds(O2, NP)])
        pltpu.sync_copy(y2_hbm, fx.at[pl.ds(O3, NP)])

        iota = lax.iota(jnp.int32, L)

        def init_v(v, _):
            o = v * L
            x1 = fx[pl.ds(o, L)]
            y1 = fx[pl.ds(O1 + o, L)]
            x2 = fx[pl.ds(O2 + o, L)]
            y2 = fx[pl.ds(O3 + o, L)]
            fx[pl.ds(O4 + o, L)] = (x2 - x1) * (y2 - y1)
            return 0
        lax.fori_loop(0, NP // L, init_v, 0)

        def init_al(v, _):
            al[pl.ds(v * L, L)] = jnp.full((L,), 1.0, jnp.float32)
            return 0
        lax.fori_loop(0, M, init_al, 0)

        def init_io(v, _):
            blkio[pl.ds(v * L, L)] = iota + v * L
            return 0
        lax.fori_loop(0, B // L, init_io, 0)

        def kblock(k, _):
            pltpu.sync_copy(al.at[pl.ds(k * L, L)], flag_sh.at[pl.ds(w * L, L)])
            plsc.subcore_barrier()
            pltpu.sync_copy(flag_sh, flagwin)
            plsc.subcore_barrier()

            kb = k * B

            def gbody(gp, _):
                o = kb + gp * L
                x1g = fx[pl.ds(o, L)]
                y1g = fx[pl.ds(O1 + o, L)]
                x2g = fx[pl.ds(O2 + o, L)]
                y2g = fx[pl.ds(O3 + o, L)]
                arg = fx[pl.ds(O4 + o, L)]
                for l in range(L):
                    alv = flagwin[pl.ds(gp * L, L)]
                    @pl.when(alv[l] > 0.5)
                    def _(l=l):
                        bx1 = x1g[l]
                        by1 = y1g[l]
                        bx2 = x2g[l]
                        by2 = y2g[l]
                        ba = arg[l]
                        p = gp * L + l

                        @plsc.parallel_loop(gp, B // L, unroll=4)
                        def vblk(v):
                            ob = kb + v * L
                            xx1 = jnp.maximum(bx1, fx[pl.ds(ob, L)])
                            yy1 = jnp.maximum(by1, fx[pl.ds(O1 + ob, L)])
                            xx2 = jnp.minimum(bx2, fx[pl.ds(O2 + ob, L)])
                            yy2 = jnp.minimum(by2, fx[pl.ds(O3 + ob, L)])
                            iw = jnp.maximum(xx2 - xx1, 0.0)
                            ih = jnp.maximum(yy2 - yy1, 0.0)
                            inter = iw * ih
                            union = jnp.maximum(ba + fx[pl.ds(O4 + ob, L)] - inter, 1e-6)
                            over = (inter > IOU_T * union) & (blkio[pl.ds(v * L, L)] > p)
                            fw = flagwin[pl.ds(v * L, L)]
                            flagwin[pl.ds(v * L, L)] = jnp.where(over, 0.0, fw)

                        @plsc.parallel_loop(k + 1, M, unroll=8)
                        def mbody(v2):
                            oc = v2 * B + w * L
                            xx1 = jnp.maximum(bx1, fx[pl.ds(oc, L)])
                            yy1 = jnp.maximum(by1, fx[pl.ds(O1 + oc, L)])
                            xx2 = jnp.minimum(bx2, fx[pl.ds(O2 + oc, L)])
                            yy2 = jnp.minimum(by2, fx[pl.ds(O3 + oc, L)])
                            iw = jnp.maximum(xx2 - xx1, 0.0)
                            ih = jnp.maximum(yy2 - yy1, 0.0)
                            inter = iw * ih
                            union = jnp.maximum(ba + fx[pl.ds(O4 + oc, L)] - inter, 1e-6)
                            over = inter > IOU_T * union
                            a = al[pl.ds(v2 * L, L)]
                            al[pl.ds(v2 * L, L)] = jnp.where(over, 0.0, a)
                return 0
            lax.fori_loop(0, B // L, gbody, 0)

            al[pl.ds(k * L, L)] = flagwin[pl.ds(w * L, L)]
            return 0

        lax.fori_loop(0, NB, kblock, 0)

        pltpu.sync_copy(sc_hbm.at[pl.ds(w * (NP // NSUB), NP // NSUB)], sco)

        def outv(v2, _):
            sco[pl.ds(v2 * L, L)] = sco[pl.ds(v2 * L, L)] * al[pl.ds(v2 * L, L)]
            return 0
        lax.fori_loop(0, M, outv, 0)
        pltpu.sync_copy(sco, out_hbm.at[pl.ds(w * (NP // NSUB), NP // NSUB)])

    return pl.kernel(
        body,
        out_type=jax.ShapeDtypeStruct((NP,), jnp.float32),
        mesh=mesh,
        scratch_types=[
            pltpu.VMEM((5 * NP,), jnp.float32),
            pltpu.VMEM((M * L,), jnp.float32),
            pltpu.VMEM((B,), jnp.float32),
            pltpu.VMEM((B,), jnp.int32),
            pltpu.VMEM((NP // NSUB,), jnp.float32),
            pltpu.VMEM_SHARED((B,), jnp.float32),
        ],
        interpret=interpret,
    )


_nms = _make_nms()


@jax.jit
def kernel(boxes, scores):
    order = jnp.argsort(-scores)
    boxes_s = jnp.take(boxes, order, axis=0)
    scores_s = jnp.take(scores, order, axis=0)
    pad = NP - N
    x1 = jnp.pad(boxes_s[:, 0], (0, pad))
    y1 = jnp.pad(boxes_s[:, 1], (0, pad))
    x2 = jnp.pad(boxes_s[:, 2], (0, pad))
    y2 = jnp.pad(boxes_s[:, 3], (0, pad))
    sc = jnp.pad(scores_s, (0, pad))
    scm = sc.reshape(NB, NSUB, L).transpose(1, 0, 2).reshape(NP)
    kept_own = _nms(x1, y1, x2, y2, scm)
    kept = kept_own.reshape(NSUB, NB, L).transpose(1, 0, 2).reshape(NP)
    return kept[:N]
